# baseline (speedup 1.0000x reference)
import functools

import jax
import jax.numpy as jnp
from jax.experimental import pallas as pl

_PRE_NMS_LIMIT = 6000
_PROPOSAL_COUNT = 1000
_NMS_THRESHOLD = 0.7
_STD = (0.1, 0.1, 0.2, 0.2)
_LANES = 128


def _proposal_kernel(s_ref, b0_ref, b1_ref, b2_ref, b3_ref,
                     a0_ref, a1_ref, a2_ref, a3_ref,
                     oy1_ref, ox1_ref, oy2_ref, ox2_ref, osc_ref):
    B, R, L = s_ref.shape
    s = s_ref[...]

    a0 = a0_ref[...]
    a1 = a1_ref[...]
    a2 = a2_ref[...]
    a3 = a3_ref[...]
    h = a2 - a0
    w = a3 - a1
    cy = a0 + 0.5 * h
    cx = a1 + 0.5 * w
    cy = cy + (b0_ref[...] * _STD[0]) * h
    cx = cx + (b1_ref[...] * _STD[1]) * w
    hh = h * jnp.exp(b2_ref[...] * _STD[2])
    ww = w * jnp.exp(b3_ref[...] * _STD[3])
    y1 = cy - 0.5 * hh
    x1 = cx - 0.5 * ww
    y2 = y1 + hh
    x2 = x1 + ww
    y1 = jnp.clip(y1, 0.0, 1.0)
    x1 = jnp.clip(x1, 0.0, 1.0)
    y2 = jnp.clip(y2, 0.0, 1.0)
    x2 = jnp.clip(x2, 0.0, 1.0)
    areas = (y2 - y1) * (x2 - x1)

    pos = (jax.lax.broadcasted_iota(jnp.int32, (B, R, L), 1) * L
           + jax.lax.broadcasted_iota(jnp.int32, (B, R, L), 2))

    def bit_step(j, cur):
        bit = jnp.left_shift(jnp.int32(1), 30 - j)
        trial = jnp.bitwise_or(cur, bit)
        x = jax.lax.bitcast_convert_type(trial, jnp.float32)
        cnt = jnp.sum(jnp.where(s >= x, 1.0, 0.0), axis=(1, 2), keepdims=True)
        return jnp.where(cnt >= jnp.float32(_PRE_NMS_LIMIT), trial, cur)

    vbits = jax.lax.fori_loop(0, 31, bit_step,
                              jnp.zeros((B, 1, 1), jnp.int32))
    v = jax.lax.bitcast_convert_type(vbits, jnp.float32)

    gt = s > v
    eq = s == v
    count_gt = jnp.sum(jnp.where(gt, 1.0, 0.0), axis=(1, 2), keepdims=True)
    need = jnp.float32(_PRE_NMS_LIMIT) - count_gt

    eqf = jnp.where(eq, 1.0, 0.0).reshape(B * R, L)
    u_in = jnp.where(jax.lax.broadcasted_iota(jnp.int32, (L, L), 0)
                     <= jax.lax.broadcasted_iota(jnp.int32, (L, L), 1),
                     1.0, 0.0)
    rowcs = jax.lax.dot_general(eqf, u_in, (((1,), (0,)), ((), ())),
                                preferred_element_type=jnp.float32)
    rowcs = rowcs.reshape(B, R, L)
    rowtot = rowcs[:, :, L - 1]
    u_ex = jnp.where(jax.lax.broadcasted_iota(jnp.int32, (R, R), 0)
                     < jax.lax.broadcasted_iota(jnp.int32, (R, R), 1),
                     1.0, 0.0)
    offs = jax.lax.dot_general(rowtot, u_ex, (((1,), (0,)), ((), ())),
                               preferred_element_type=jnp.float32)
    cs = rowcs + offs[:, :, None]

    zero_out = jnp.zeros((_PROPOSAL_COUNT, B), jnp.float32)
    oy1_ref[...] = zero_out
    ox1_ref[...] = zero_out
    oy2_ref[...] = zero_out
    ox2_ref[...] = zero_out
    osc_ref[...] = zero_out

    NEG = jnp.float32(-1e30)
    BIGI = jnp.int32(2 ** 30)

    ms0 = jnp.where(gt | (eq & (cs <= need)), s, NEG)

    def nms_step(i, ms):
        m1 = jnp.max(ms, axis=(1, 2), keepdims=True)
        has = m1 > jnp.float32(-1e29)
        cand = jnp.where(ms == m1, pos, BIGI)
        idx = jnp.min(cand, axis=(1, 2), keepdims=True)
        one = pos == idx
        by1 = jnp.sum(jnp.where(one, y1, 0.0), axis=(1, 2), keepdims=True)
        bx1 = jnp.sum(jnp.where(one, x1, 0.0), axis=(1, 2), keepdims=True)
        by2 = jnp.sum(jnp.where(one, y2, 0.0), axis=(1, 2), keepdims=True)
        bx2 = jnp.sum(jnp.where(one, x2, 0.0), axis=(1, 2), keepdims=True)

        yy1 = jnp.maximum(by1, y1)
        xx1 = jnp.maximum(bx1, x1)
        yy2 = jnp.minimum(by2, y2)
        xx2 = jnp.minimum(bx2, x2)
        inter = (jnp.maximum(yy2 - yy1, 0.0) * jnp.maximum(xx2 - xx1, 0.0))
        barea = (by2 - by1) * (bx2 - bx1)
        union = barea + areas - inter
        iou = inter / jnp.maximum(union, 1e-8)
        suppress = has & ((iou > _NMS_THRESHOLD) | one)
        ms = jnp.where(suppress, NEG, ms)

        oy1_ref[pl.ds(i, 1), :] = jnp.where(has, by1, 0.0).reshape(1, B)
        ox1_ref[pl.ds(i, 1), :] = jnp.where(has, bx1, 0.0).reshape(1, B)
        oy2_ref[pl.ds(i, 1), :] = jnp.where(has, by2, 0.0).reshape(1, B)
        ox2_ref[pl.ds(i, 1), :] = jnp.where(has, bx2, 0.0).reshape(1, B)
        osc_ref[pl.ds(i, 1), :] = jnp.where(has, m1, 0.0).reshape(1, B)
        return ms

    jax.lax.fori_loop(0, _PROPOSAL_COUNT, nms_step, ms0)


@functools.partial(jax.jit)
def kernel(rpn_probs, rpn_bbox, anchors):
    B, N, _ = rpn_probs.shape
    R = (N + _LANES - 1) // _LANES
    R = ((R + 7) // 8) * 8
    pad = R * _LANES - N

    def prep(x, fill):
        return jnp.pad(x, ((0, 0), (0, pad)),
                       constant_values=fill).reshape(B, R, _LANES)

    s = prep(rpn_probs[:, :, 1], -1.0)
    bb = [prep(rpn_bbox[:, :, k], 0.0) for k in range(4)]
    aa = [prep(anchors[:, :, k], 0.0) for k in range(4)]

    out_sds = [jax.ShapeDtypeStruct((_PROPOSAL_COUNT, B), jnp.float32)] * 5
    oy1, ox1, oy2, ox2, osc = pl.pallas_call(
        _proposal_kernel,
        out_shape=out_sds,
    )(s, *bb, *aa)

    proposals = jnp.stack([oy1.T, ox1.T, oy2.T, ox2.T], axis=-1)
    return proposals, osc.T


# speedup vs baseline: 1.5048x; 1.5048x over previous
import functools

import jax
import jax.numpy as jnp
from jax import lax
from jax.experimental import pallas as pl
from jax.experimental.pallas import tpu as pltpu, tpu_sc as plsc

_PRE_NMS_LIMIT = 6000
_PROPOSAL_COUNT = 1000
_NMS_THRESHOLD = 0.7
_STD = (0.1, 0.1, 0.2, 0.2)
_LANES = 128

_B = 4
_R = 160
_RL = _R * _LANES
_CHUNKS = 8
_CHUNK = _RL // _CHUNKS
_CR = 48
_C = _CR * _LANES


def _prep_kernel(s_ref, b0_ref, b1_ref, b2_ref, b3_ref,
                 a0_ref, a1_ref, a2_ref, a3_ref,
                 y1_ref, x1_ref, y2_ref, x2_ref, sc_ref, dst_ref):
    B, R, L = s_ref.shape
    s = s_ref[...]

    a0 = a0_ref[...]
    a1 = a1_ref[...]
    a2 = a2_ref[...]
    a3 = a3_ref[...]
    h = a2 - a0
    w = a3 - a1
    cy = a0 + 0.5 * h
    cx = a1 + 0.5 * w
    cy = cy + (b0_ref[...] * _STD[0]) * h
    cx = cx + (b1_ref[...] * _STD[1]) * w
    hh = h * jnp.exp(b2_ref[...] * _STD[2])
    ww = w * jnp.exp(b3_ref[...] * _STD[3])
    y1 = cy - 0.5 * hh
    x1 = cx - 0.5 * ww
    y2 = y1 + hh
    x2 = x1 + ww
    y1_ref[...] = jnp.clip(y1, 0.0, 1.0)
    x1_ref[...] = jnp.clip(x1, 0.0, 1.0)
    y2_ref[...] = jnp.clip(y2, 0.0, 1.0)
    x2_ref[...] = jnp.clip(x2, 0.0, 1.0)
    sc_ref[...] = s

    def bit_step(j, cur):
        bit = jnp.left_shift(jnp.int32(1), 30 - j)
        trial = jnp.bitwise_or(cur, bit)
        x = lax.bitcast_convert_type(trial, jnp.float32)
        cnt = jnp.sum(jnp.where(s >= x, 1.0, 0.0), axis=(1, 2), keepdims=True)
        return jnp.where(cnt >= jnp.float32(_PRE_NMS_LIMIT), trial, cur)

    vbits = lax.fori_loop(0, 31, bit_step, jnp.zeros((B, 1, 1), jnp.int32))
    v = lax.bitcast_convert_type(vbits, jnp.float32)

    gt = s > v
    eq = s == v
    count_gt = jnp.sum(jnp.where(gt, 1.0, 0.0), axis=(1, 2), keepdims=True)
    need = jnp.float32(_PRE_NMS_LIMIT) - count_gt

    u_in = jnp.where(lax.broadcasted_iota(jnp.int32, (L, L), 0)
                     <= lax.broadcasted_iota(jnp.int32, (L, L), 1),
                     1.0, 0.0)
    u_ex = jnp.where(lax.broadcasted_iota(jnp.int32, (R, R), 0)
                     < lax.broadcasted_iota(jnp.int32, (R, R), 1),
                     1.0, 0.0)

    def prefix(maskf):
        flat = maskf.reshape(B * R, L)
        rowcs = lax.dot_general(flat, u_in, (((1,), (0,)), ((), ())),
                                preferred_element_type=jnp.float32)
        rowcs = rowcs.reshape(B, R, L)
        rowtot = rowcs[:, :, L - 1]
        offs = lax.dot_general(rowtot, u_ex, (((1,), (0,)), ((), ())),
                               preferred_element_type=jnp.float32)
        return rowcs + offs[:, :, None]

    cs_eq = prefix(jnp.where(eq, 1.0, 0.0))
    valid = gt | (eq & (cs_eq <= need))
    cs_v = prefix(jnp.where(valid, 1.0, 0.0))
    dst_ref[...] = jnp.where(valid, cs_v.astype(jnp.int32) - 1, -1)


def _make_compact_kernel():
    mesh = plsc.VectorSubcoreMesh(core_axis_name="c", subcore_axis_name="s")
    out_t = [jax.ShapeDtypeStruct((_B, _CHUNKS, _C), jnp.float32)] * 5
    scratch = ([pltpu.VMEM((_CHUNK,), jnp.int32)]
               + [pltpu.VMEM((_CHUNK,), jnp.float32)] * 5
               + [pltpu.VMEM((_C,), jnp.float32)] * 5)

    @functools.partial(pl.kernel, mesh=mesh, out_type=out_t,
                       scratch_types=scratch,
                       compiler_params=pltpu.CompilerParams(
                           needs_layout_passes=False))
    def compact(dst_hbm, py1, px1, py2, px2, psc,
                oy1, ox1, oy2, ox2, osc,
                idx_v, sy1, sx1, sy2, sx2, ssc,
                dy1, dx1, dy2, dx2, dsc):
        wid = lax.axis_index("s") * 2 + lax.axis_index("c")
        b = wid // _CHUNKS
        ch = wid % _CHUNKS
        base = ch * _CHUNK

        pltpu.sync_copy(dst_hbm.at[b, pl.ds(base, _CHUNK)], idx_v)
        pltpu.sync_copy(py1.at[b, pl.ds(base, _CHUNK)], sy1)
        pltpu.sync_copy(px1.at[b, pl.ds(base, _CHUNK)], sx1)
        pltpu.sync_copy(py2.at[b, pl.ds(base, _CHUNK)], sy2)
        pltpu.sync_copy(px2.at[b, pl.ds(base, _CHUNK)], sx2)
        pltpu.sync_copy(psc.at[b, pl.ds(base, _CHUNK)], ssc)

        zeros16 = jnp.zeros((16,), jnp.float32)

        def zero_body(k, _):
            sl = pl.ds(k * 16, 16)
            dy1[sl] = zeros16
            dx1[sl] = zeros16
            dy2[sl] = zeros16
            dx2[sl] = zeros16
            dsc[sl] = zeros16
            return 0

        lax.fori_loop(0, _C // 16, zero_body, 0)

        def scat_body(j, _):
            sl = pl.ds(j * 16, 16)
            idx16 = idx_v[sl]
            m = idx16 >= 0
            plsc.store_scatter(dy1, [idx16], sy1[sl], mask=m)
            plsc.store_scatter(dx1, [idx16], sx1[sl], mask=m)
            plsc.store_scatter(dy2, [idx16], sy2[sl], mask=m)
            plsc.store_scatter(dx2, [idx16], sx2[sl], mask=m)
            plsc.store_scatter(dsc, [idx16], ssc[sl], mask=m)
            return 0

        lax.fori_loop(0, _CHUNK // 16, scat_body, 0)

        pltpu.sync_copy(dy1, oy1.at[b, ch])
        pltpu.sync_copy(dx1, ox1.at[b, ch])
        pltpu.sync_copy(dy2, oy2.at[b, ch])
        pltpu.sync_copy(dx2, ox2.at[b, ch])
        pltpu.sync_copy(dsc, osc.at[b, ch])

    return compact


_compact_call = _make_compact_kernel()


def _nms_kernel(y1_ref, x1_ref, y2_ref, x2_ref, s_ref,
                oy1_ref, ox1_ref, oy2_ref, ox2_ref, osc_ref):
    B = y1_ref.shape[0]

    def merge(ref):
        x = ref[...]
        acc = x[:, 0]
        for wv in range(1, _CHUNKS):
            acc = acc + x[:, wv]
        return acc

    y1 = merge(y1_ref)
    x1 = merge(x1_ref)
    y2 = merge(y2_ref)
    x2 = merge(x2_ref)
    s = merge(s_ref)
    areas = (y2 - y1) * (x2 - x1)

    pos = (lax.broadcasted_iota(jnp.int32, (B, _CR, _LANES), 1) * _LANES
           + lax.broadcasted_iota(jnp.int32, (B, _CR, _LANES), 2))

    zero_out = jnp.zeros((_PROPOSAL_COUNT, B), jnp.float32)
    oy1_ref[...] = zero_out
    ox1_ref[...] = zero_out
    oy2_ref[...] = zero_out
    ox2_ref[...] = zero_out
    osc_ref[...] = zero_out

    NEG = jnp.float32(-1e30)
    BIGI = jnp.int32(2 ** 30)
    ms0 = jnp.where(pos < _PRE_NMS_LIMIT, s, NEG)

    def nms_step(i, ms):
        m1 = jnp.max(ms, axis=(1, 2), keepdims=True)
        has = m1 > jnp.float32(-1e29)
        cand = jnp.where(ms == m1, pos, BIGI)
        idx = jnp.min(cand, axis=(1, 2), keepdims=True)
        one = pos == idx
        by1 = jnp.sum(jnp.where(one, y1, 0.0), axis=(1, 2), keepdims=True)
        bx1 = jnp.sum(jnp.where(one, x1, 0.0), axis=(1, 2), keepdims=True)
        by2 = jnp.sum(jnp.where(one, y2, 0.0), axis=(1, 2), keepdims=True)
        bx2 = jnp.sum(jnp.where(one, x2, 0.0), axis=(1, 2), keepdims=True)

        yy1 = jnp.maximum(by1, y1)
        xx1 = jnp.maximum(bx1, x1)
        yy2 = jnp.minimum(by2, y2)
        xx2 = jnp.minimum(bx2, x2)
        inter = (jnp.maximum(yy2 - yy1, 0.0) * jnp.maximum(xx2 - xx1, 0.0))
        barea = (by2 - by1) * (bx2 - bx1)
        union = barea + areas - inter
        iou = inter / jnp.maximum(union, 1e-8)
        suppress = has & ((iou > _NMS_THRESHOLD) | one)
        ms = jnp.where(suppress, NEG, ms)

        oy1_ref[pl.ds(i, 1), :] = jnp.where(has, by1, 0.0).reshape(1, B)
        ox1_ref[pl.ds(i, 1), :] = jnp.where(has, bx1, 0.0).reshape(1, B)
        oy2_ref[pl.ds(i, 1), :] = jnp.where(has, by2, 0.0).reshape(1, B)
        ox2_ref[pl.ds(i, 1), :] = jnp.where(has, bx2, 0.0).reshape(1, B)
        osc_ref[pl.ds(i, 1), :] = jnp.where(has, m1, 0.0).reshape(1, B)
        return ms

    lax.fori_loop(0, _PROPOSAL_COUNT, nms_step, ms0)


@jax.jit
def kernel(rpn_probs, rpn_bbox, anchors):
    B, N, _ = rpn_probs.shape
    pad = _RL - N

    def prep(x, fill):
        return jnp.pad(x, ((0, 0), (0, pad)),
                       constant_values=fill).reshape(B, _R, _LANES)

    s = prep(rpn_probs[:, :, 1], -1.0)
    bb = [prep(rpn_bbox[:, :, k], 0.0) for k in range(4)]
    aa = [prep(anchors[:, :, k], 0.0) for k in range(4)]

    plane_t = jax.ShapeDtypeStruct((B, _R, _LANES), jnp.float32)
    y1p, x1p, y2p, x2p, scp, dstp = pl.pallas_call(
        _prep_kernel,
        out_shape=[plane_t] * 5 + [jax.ShapeDtypeStruct((B, _R, _LANES),
                                                        jnp.int32)],
    )(s, *bb, *aa)

    flat = lambda a: a.reshape(B, _RL)
    cy1, cx1, cy2, cx2, csc = _compact_call(
        flat(dstp), flat(y1p), flat(x1p), flat(y2p), flat(x2p), flat(scp))

    shaped = lambda a: a.reshape(B, _CHUNKS, _CR, _LANES)
    out_sds = [jax.ShapeDtypeStruct((_PROPOSAL_COUNT, B), jnp.float32)] * 5
    oy1, ox1, oy2, ox2, osc = pl.pallas_call(
        _nms_kernel,
        out_shape=out_sds,
    )(shaped(cy1), shaped(cx1), shaped(cy2), shaped(cx2), shaped(csc))

    proposals = jnp.stack([oy1.T, ox1.T, oy2.T, ox2.T], axis=-1)
    return proposals, osc.T
